# baseline (speedup 1.0000x reference)
import functools

import jax
import jax.numpy as jnp
from jax import lax
from jax.experimental import pallas as pl
from jax.experimental.pallas import tpu as pltpu
from jax.experimental.pallas import tpu_sc as plsc

B = 16384
D = 64
NW = 32
BPW = B // NW
CH = 128
NCH = BPW // CH
G = 16

_mesh = plsc.VectorSubcoreMesh(core_axis_name="c", subcore_axis_name="s")


@functools.partial(
    pl.kernel,
    mesh=_mesh,
    out_type=jax.ShapeDtypeStruct((B,), jnp.float32),
    compiler_params=pltpu.CompilerParams(use_tc_tiling_on_sc=False),
    scratch_types=[
        pltpu.VMEM((BPW,), jnp.int32),
        pltpu.VMEM((BPW,), jnp.int32),
        pltpu.VMEM((BPW,), jnp.int32),
        pltpu.VMEM((CH, D), jnp.float32),
        pltpu.VMEM((CH, D), jnp.float32),
        pltpu.VMEM((CH, D), jnp.float32),
        pltpu.VMEM((CH, D), jnp.float32),
        pltpu.VMEM((CH, D), jnp.float32),
        pltpu.VMEM((CH, D), jnp.float32),
        pltpu.VMEM((BPW,), jnp.float32),
        pltpu.SemaphoreType.DMA,
    ],
)
def _kg_score(heads, relations, tails, er, ei, rrt, rit, out,
              hidx, ridx, tidx, hr, hi, tr, ti, rr, ri, outv, sem):
    wid = lax.axis_index("s") * 2 + lax.axis_index("c")
    base = wid * BPW
    pltpu.sync_copy(heads.at[pl.ds(base, BPW)], hidx)
    pltpu.sync_copy(relations.at[pl.ds(base, BPW)], ridx)
    pltpu.sync_copy(tails.at[pl.ds(base, BPW)], tidx)

    lanes = lax.iota(jnp.int32, G)
    perms = [lanes ^ s for s in (8, 4, 2, 1)]

    for j in range(NCH):
        c0 = j * CH
        cps = [
            pltpu.async_copy(er.at[hidx.at[pl.ds(c0, CH)]], hr, sem),
            pltpu.async_copy(ei.at[hidx.at[pl.ds(c0, CH)]], hi, sem),
            pltpu.async_copy(er.at[tidx.at[pl.ds(c0, CH)]], tr, sem),
            pltpu.async_copy(ei.at[tidx.at[pl.ds(c0, CH)]], ti, sem),
            pltpu.async_copy(rrt.at[ridx.at[pl.ds(c0, CH)]], rr, sem),
            pltpu.async_copy(rit.at[ridx.at[pl.ds(c0, CH)]], ri, sem),
        ]
        for cp in cps:
            cp.wait()

        def group_body(g, carry):
            i0 = g * G
            res = jnp.zeros((G,), jnp.float32)
            for li in range(G):
                i = i0 + li
                acc = jnp.zeros((G,), jnp.float32)
                for k in range(D // G):
                    sl = pl.ds(k * G, G)
                    a = hr[i, sl]
                    b = hi[i, sl]
                    c = rr[i, sl]
                    d = ri[i, sl]
                    e = tr[i, sl]
                    f = ti[i, sl]
                    acc = acc + (a * c + b * d) * e + (a * d - b * c) * f
                for p in perms:
                    acc = acc + jnp.take(acc, p)
                res = jnp.where(lanes == li, acc, res)
            outv[pl.ds(c0 + i0, G)] = res
            return carry

        lax.fori_loop(0, CH // G, group_body, 0)

    pltpu.sync_copy(outv, out.at[pl.ds(base, BPW)])


def kernel(heads, relations, tails, entity_real, entity_imag,
           rel_real, rel_imag):
    return _kg_score(heads.astype(jnp.int32), relations.astype(jnp.int32),
                     tails.astype(jnp.int32), entity_real, entity_imag,
                     rel_real, rel_imag)
